# Initial kernel scaffold; baseline (speedup 1.0000x reference)
import jax
import jax.numpy as jnp
from jax.experimental import pallas as pl


def kernel(x, edge_index, W1, b1, W2, b2):
    raise NotImplementedError("write your pallas kernel here")


# speedup vs baseline: 31.2158x; 31.2158x over previous
import functools

import jax
import jax.numpy as jnp
from jax import lax
from jax.experimental import pallas as pl
from jax.experimental.pallas import tpu as pltpu
from jax.experimental.pallas import tpu_sc as plsc

N = 10000
NP = 10240
E = 320000
DF = 128
DH = 64
DO = 16
NC, NS = 2, 16
NW = NC * NS
ER = 2560
RPW = ER // NW
CPT = NP // NS

_SC_PARAMS = pltpu.CompilerParams(use_tc_tiling_on_sc=False)


def _mesh():
    return plsc.VectorSubcoreMesh(
        core_axis_name="c", subcore_axis_name="s", num_cores=NC, num_subcores=NS
    )


def _deg_partials(dst2):
    @functools.partial(
        pl.kernel,
        out_type=(
            jax.ShapeDtypeStruct((NP,), jnp.float32),
            jax.ShapeDtypeStruct((NP,), jnp.float32),
        ),
        mesh=_mesh(),
        scratch_types=[
            pltpu.VMEM((RPW, 128), jnp.int32),
            pltpu.VMEM((128,), jnp.float32),
            pltpu.VMEM((CPT,), jnp.float32),
            pltpu.VMEM_SHARED((NP,), jnp.float32),
        ],
        compiler_params=_SC_PARAMS,
    )
    def kern(dst_hbm, out0_hbm, out1_hbm, didx_v, ones_v, zer_v, deg_sp):
        c = lax.axis_index("c")
        s = lax.axis_index("s")
        w = s * NC + c

        def fill1(i, _):
            ones_v[pl.ds(i * 16, 16)] = jnp.ones((16,), jnp.float32)
            return 0

        lax.fori_loop(0, 128 // 16, fill1, 0)

        def fill0(i, _):
            zer_v[pl.ds(i * 16, 16)] = jnp.zeros((16,), jnp.float32)
            return 0

        lax.fori_loop(0, CPT // 16, fill0, 0)
        pltpu.sync_copy(zer_v, deg_sp.at[pl.ds(s * CPT, CPT)])
        plsc.subcore_barrier()

        pltpu.sync_copy(dst_hbm.at[pl.ds(w * RPW, RPW)], didx_v)

        def body(j, _):
            pltpu.sync_copy(ones_v, deg_sp.at[didx_v.at[j]], add=True)
            return 0

        lax.fori_loop(0, RPW, body, 0)
        plsc.subcore_barrier()

        @pl.when(c == 0)
        def _():
            pltpu.sync_copy(deg_sp.at[pl.ds(s * CPT, CPT)], out0_hbm.at[pl.ds(s * CPT, CPT)])

        @pl.when(c == 1)
        def _():
            pltpu.sync_copy(deg_sp.at[pl.ds(s * CPT, CPT)], out1_hbm.at[pl.ds(s * CPT, CPT)])

    return kern(dst2)


def _scatter_partials(g, src2, dst2, d):
    @functools.partial(
        pl.kernel,
        out_type=(
            jax.ShapeDtypeStruct((NP, d), jnp.float32),
            jax.ShapeDtypeStruct((NP, d), jnp.float32),
        ),
        mesh=_mesh(),
        scratch_types=[
            pltpu.VMEM((RPW, 128), jnp.int32),
            pltpu.VMEM((RPW, 128), jnp.int32),
            pltpu.VMEM((128, d), jnp.float32),
            pltpu.VMEM((64, d), jnp.float32),
            pltpu.VMEM_SHARED((NP, d), jnp.float32),
        ],
        compiler_params=_SC_PARAMS,
    )
    def kern(g_hbm, src_hbm, dst_hbm, out0_hbm, out1_hbm,
             sidx_v, didx_v, rows_v, zer_v, acc_sp):
        c = lax.axis_index("c")
        s = lax.axis_index("s")
        w = s * NC + c

        def fill0(i, _):
            for jj in range(d // 16):
                zer_v[i, pl.ds(jj * 16, 16)] = jnp.zeros((16,), jnp.float32)
            return 0

        lax.fori_loop(0, 64, fill0, 0)

        def zcp(i, _):
            pltpu.sync_copy(zer_v, acc_sp.at[pl.ds(s * CPT + i * 64, 64)])
            return 0

        lax.fori_loop(0, CPT // 64, zcp, 0)
        plsc.subcore_barrier()

        pltpu.sync_copy(src_hbm.at[pl.ds(w * RPW, RPW)], sidx_v)
        pltpu.sync_copy(dst_hbm.at[pl.ds(w * RPW, RPW)], didx_v)

        def body(j, _):
            pltpu.sync_copy(g_hbm.at[sidx_v.at[j]], rows_v)
            pltpu.sync_copy(rows_v, acc_sp.at[didx_v.at[j]], add=True)
            return 0

        lax.fori_loop(0, RPW, body, 0)
        plsc.subcore_barrier()

        @pl.when(c == 0)
        def _():
            pltpu.sync_copy(acc_sp.at[pl.ds(s * CPT, CPT)], out0_hbm.at[pl.ds(s * CPT, CPT)])

        @pl.when(c == 1)
        def _():
            pltpu.sync_copy(acc_sp.at[pl.ds(s * CPT, CPT)], out1_hbm.at[pl.ds(s * CPT, CPT)])

    return kern(g, src2, dst2)


def _dinv(d0_ref, d1_ref):
    return lax.rsqrt(d0_ref[...] + d1_ref[...] + 1.0)


def _tc1_body(x_ref, w_ref, d0_ref, d1_ref, o_ref):
    dinv = _dinv(d0_ref, d1_ref)
    o_ref[...] = dinv * jnp.dot(
        x_ref[...], w_ref[...], preferred_element_type=jnp.float32
    )


def _tc2_body(p0_ref, p1_ref, g_ref, d0_ref, d1_ref, w_ref, b_ref, o_ref):
    dinv = _dinv(d0_ref, d1_ref)
    a = p0_ref[...] + p1_ref[...] + g_ref[...]
    t = jnp.maximum(dinv * a + b_ref[...], 0.0)
    o_ref[...] = dinv * jnp.dot(t, w_ref[...], preferred_element_type=jnp.float32)


def _tc3_body(p0_ref, p1_ref, g_ref, d0_ref, d1_ref, b_ref, o_ref):
    dinv = _dinv(d0_ref, d1_ref)
    o = dinv * (p0_ref[...] + p1_ref[...] + g_ref[...]) + b_ref[...]
    m = jnp.max(o, axis=1, keepdims=True)
    ex = jnp.exp(o - m)
    lse = jnp.log(jnp.sum(ex, axis=1, keepdims=True))
    o_ref[...] = o - m - lse


_BLK = 1024


def _col_spec(i):
    return (i, 0)


def _tc1(xp, W1, d0, d1):
    return pl.pallas_call(
        _tc1_body,
        grid=(NP // _BLK,),
        in_specs=[
            pl.BlockSpec((_BLK, DF), _col_spec),
            pl.BlockSpec((DF, DH), lambda i: (0, 0)),
            pl.BlockSpec((_BLK, 1), _col_spec),
            pl.BlockSpec((_BLK, 1), _col_spec),
        ],
        out_specs=pl.BlockSpec((_BLK, DH), _col_spec),
        out_shape=jax.ShapeDtypeStruct((NP, DH), jnp.float32),
    )(xp, W1, d0, d1)


def _tc2(p0, p1, g1, d0, d1, W2, b1r):
    return pl.pallas_call(
        _tc2_body,
        grid=(NP // _BLK,),
        in_specs=[
            pl.BlockSpec((_BLK, DH), _col_spec),
            pl.BlockSpec((_BLK, DH), _col_spec),
            pl.BlockSpec((_BLK, DH), _col_spec),
            pl.BlockSpec((_BLK, 1), _col_spec),
            pl.BlockSpec((_BLK, 1), _col_spec),
            pl.BlockSpec((DH, DO), lambda i: (0, 0)),
            pl.BlockSpec((1, DH), lambda i: (0, 0)),
        ],
        out_specs=pl.BlockSpec((_BLK, DO), _col_spec),
        out_shape=jax.ShapeDtypeStruct((NP, DO), jnp.float32),
    )(p0, p1, g1, d0, d1, W2, b1r)


def _tc3(p0, p1, g2, d0, d1, b2r):
    return pl.pallas_call(
        _tc3_body,
        grid=(NP // _BLK,),
        in_specs=[
            pl.BlockSpec((_BLK, DO), _col_spec),
            pl.BlockSpec((_BLK, DO), _col_spec),
            pl.BlockSpec((_BLK, DO), _col_spec),
            pl.BlockSpec((_BLK, 1), _col_spec),
            pl.BlockSpec((_BLK, 1), _col_spec),
            pl.BlockSpec((1, DO), lambda i: (0, 0)),
        ],
        out_specs=pl.BlockSpec((_BLK, DO), _col_spec),
        out_shape=jax.ShapeDtypeStruct((NP, DO), jnp.float32),
    )(p0, p1, g2, d0, d1, b2r)


def kernel(x, edge_index, W1, b1, W2, b2):
    padn = ER * 128 - E
    pad = (N + (jnp.arange(padn, dtype=jnp.int32) % (NP - N))).astype(jnp.int32)
    src2 = jnp.concatenate([edge_index[0], pad]).reshape(ER, 128)
    dst2 = jnp.concatenate([edge_index[1], pad]).reshape(ER, 128)

    deg0, deg1 = _deg_partials(dst2)
    d0 = deg0.reshape(NP, 1)
    d1 = deg1.reshape(NP, 1)

    xp = jnp.zeros((NP, DF), jnp.float32).at[:N].set(x)
    g1 = _tc1(xp, W1, d0, d1)
    p0, p1 = _scatter_partials(g1, src2, dst2, DH)
    g2 = _tc2(p0, p1, g1, d0, d1, W2, b1.reshape(1, DH))
    q0, q1 = _scatter_partials(g2, src2, dst2, DO)
    out = _tc3(q0, q1, g2, d0, d1, b2.reshape(1, DO))
    return out[:N]


# speedup vs baseline: 36.6325x; 1.1735x over previous
import functools

import jax
import jax.numpy as jnp
from jax import lax
from jax.experimental import pallas as pl
from jax.experimental.pallas import tpu as pltpu
from jax.experimental.pallas import tpu_sc as plsc

N = 10000
NP = 10240
E = 320000
DF = 128
DH = 64
DO = 16
NC, NS = 2, 16
NW = NC * NS
ER = 2560
RPW = ER // NW
CPT = NP // NS

_SC_PARAMS = pltpu.CompilerParams(use_tc_tiling_on_sc=False)


def _mesh():
    return plsc.VectorSubcoreMesh(
        core_axis_name="c", subcore_axis_name="s", num_cores=NC, num_subcores=NS
    )


def _deg_partials(dst2):
    @functools.partial(
        pl.kernel,
        out_type=(
            jax.ShapeDtypeStruct((NP,), jnp.float32),
            jax.ShapeDtypeStruct((NP,), jnp.float32),
        ),
        mesh=_mesh(),
        scratch_types=[
            pltpu.VMEM((RPW, 128), jnp.int32),
            pltpu.VMEM((128,), jnp.float32),
            pltpu.VMEM((CPT,), jnp.float32),
            pltpu.VMEM_SHARED((NP,), jnp.float32),
            pltpu.SemaphoreType.DMA,
        ],
        compiler_params=_SC_PARAMS,
    )
    def kern(dst_hbm, out0_hbm, out1_hbm, didx_v, ones_v, zer_v, deg_sp, sem):
        c = lax.axis_index("c")
        s = lax.axis_index("s")
        w = s * NC + c

        def fill1(i, _):
            ones_v[pl.ds(i * 16, 16)] = jnp.ones((16,), jnp.float32)
            return 0

        lax.fori_loop(0, 128 // 16, fill1, 0)

        def fill0(i, _):
            zer_v[pl.ds(i * 16, 16)] = jnp.zeros((16,), jnp.float32)
            return 0

        lax.fori_loop(0, CPT // 16, fill0, 0)
        pltpu.sync_copy(zer_v, deg_sp.at[pl.ds(s * CPT, CPT)])
        plsc.subcore_barrier()

        pltpu.sync_copy(dst_hbm.at[pl.ds(w * RPW, RPW)], didx_v)

        def body(j, _):
            pltpu.async_copy(ones_v, deg_sp.at[didx_v.at[j]], sem, add=True)
            return 0

        lax.fori_loop(0, RPW, body, 0)

        def drain(j, _):
            pltpu.make_async_copy(ones_v, deg_sp.at[didx_v.at[j]], sem).wait()
            return 0

        lax.fori_loop(0, RPW, drain, 0)
        plsc.subcore_barrier()

        @pl.when(c == 0)
        def _():
            pltpu.sync_copy(deg_sp.at[pl.ds(s * CPT, CPT)], out0_hbm.at[pl.ds(s * CPT, CPT)])

        @pl.when(c == 1)
        def _():
            pltpu.sync_copy(deg_sp.at[pl.ds(s * CPT, CPT)], out1_hbm.at[pl.ds(s * CPT, CPT)])

    return kern(dst2)


def _scatter_partials(g, src2, dst2, d):
    @functools.partial(
        pl.kernel,
        out_type=(
            jax.ShapeDtypeStruct((NP, d), jnp.float32),
            jax.ShapeDtypeStruct((NP, d), jnp.float32),
        ),
        mesh=_mesh(),
        scratch_types=[
            pltpu.VMEM((RPW, 128), jnp.int32),
            pltpu.VMEM((RPW, 128), jnp.int32),
            pltpu.VMEM((128, d), jnp.float32),
            pltpu.VMEM((128, d), jnp.float32),
            pltpu.VMEM((64, d), jnp.float32),
            pltpu.VMEM_SHARED((NP, d), jnp.float32),
            pltpu.SemaphoreType.DMA,
            pltpu.SemaphoreType.DMA,
            pltpu.SemaphoreType.DMA,
            pltpu.SemaphoreType.DMA,
            pltpu.SemaphoreType.DMA,
        ],
        compiler_params=_SC_PARAMS,
    )
    def kern(g_hbm, src_hbm, dst_hbm, out0_hbm, out1_hbm,
             sidx_v, didx_v, rows0_v, rows1_v, zer_v, acc_sp,
             semg0, semg1, sems0, sems1, semi):
        c = lax.axis_index("c")
        s = lax.axis_index("s")
        w = s * NC + c

        pltpu.async_copy(src_hbm.at[pl.ds(w * RPW, RPW)], sidx_v, semi)
        pltpu.async_copy(dst_hbm.at[pl.ds(w * RPW, RPW)], didx_v, semi)

        def fill0(i, _):
            for jj in range(d // 16):
                zer_v[i, pl.ds(jj * 16, 16)] = jnp.zeros((16,), jnp.float32)
            return 0

        lax.fori_loop(0, 64, fill0, 0)

        def zcp(i, _):
            pltpu.sync_copy(zer_v, acc_sp.at[pl.ds(s * CPT + i * 64, 64)])
            return 0

        lax.fori_loop(0, CPT // 64, zcp, 0)
        pltpu.make_async_copy(src_hbm.at[pl.ds(w * RPW, RPW)], sidx_v, semi).wait()
        pltpu.make_async_copy(dst_hbm.at[pl.ds(w * RPW, RPW)], didx_v, semi).wait()
        plsc.subcore_barrier()

        def gat(j, rows, sem):
            pltpu.async_copy(g_hbm.at[sidx_v.at[j]], rows, sem)

        def gat_wait(j, rows, sem):
            pltpu.make_async_copy(g_hbm.at[sidx_v.at[j]], rows, sem).wait()

        def sca(j, rows, sem):
            pltpu.async_copy(rows, acc_sp.at[didx_v.at[j]], sem, add=True)

        def sca_wait(j, rows, sem):
            pltpu.make_async_copy(rows, acc_sp.at[didx_v.at[j]], sem).wait()

        gat(0, rows0_v, semg0)

        def body(i, _):
            j0 = 2 * i
            j1 = 2 * i + 1
            gat_wait(j0, rows0_v, semg0)

            @pl.when(i > 0)
            def _():
                sca_wait(j1 - 2, rows1_v, sems1)

            gat(j1, rows1_v, semg1)
            sca(j0, rows0_v, sems0)
            gat_wait(j1, rows1_v, semg1)

            @pl.when(i < RPW // 2 - 1)
            def _():
                sca_wait(j0, rows0_v, sems0)
                gat(j0 + 2, rows0_v, semg0)

            sca(j1, rows1_v, sems1)
            return 0

        lax.fori_loop(0, RPW // 2, body, 0)
        sca_wait(RPW - 2, rows0_v, sems0)
        sca_wait(RPW - 1, rows1_v, sems1)
        plsc.subcore_barrier()

        @pl.when(c == 0)
        def _():
            pltpu.sync_copy(acc_sp.at[pl.ds(s * CPT, CPT)], out0_hbm.at[pl.ds(s * CPT, CPT)])

        @pl.when(c == 1)
        def _():
            pltpu.sync_copy(acc_sp.at[pl.ds(s * CPT, CPT)], out1_hbm.at[pl.ds(s * CPT, CPT)])

    return kern(g, src2, dst2)


def _dinv(d0_ref, d1_ref):
    return lax.rsqrt(d0_ref[...] + d1_ref[...] + 1.0)


def _tc1_body(x_ref, w_ref, d0_ref, d1_ref, o_ref):
    dinv = _dinv(d0_ref, d1_ref)
    o_ref[...] = dinv * jnp.dot(
        x_ref[...], w_ref[...], preferred_element_type=jnp.float32
    )


def _tc2_body(p0_ref, p1_ref, g_ref, d0_ref, d1_ref, w_ref, b_ref, o_ref):
    dinv = _dinv(d0_ref, d1_ref)
    a = p0_ref[...] + p1_ref[...] + g_ref[...]
    t = jnp.maximum(dinv * a + b_ref[...], 0.0)
    o_ref[...] = dinv * jnp.dot(t, w_ref[...], preferred_element_type=jnp.float32)


def _tc3_body(p0_ref, p1_ref, g_ref, d0_ref, d1_ref, b_ref, o_ref):
    dinv = _dinv(d0_ref, d1_ref)
    o = dinv * (p0_ref[...] + p1_ref[...] + g_ref[...]) + b_ref[...]
    m = jnp.max(o, axis=1, keepdims=True)
    ex = jnp.exp(o - m)
    lse = jnp.log(jnp.sum(ex, axis=1, keepdims=True))
    o_ref[...] = o - m - lse


_BLK = 1024


def _col_spec(i):
    return (i, 0)


def _tc1(xp, W1, d0, d1):
    return pl.pallas_call(
        _tc1_body,
        grid=(NP // _BLK,),
        in_specs=[
            pl.BlockSpec((_BLK, DF), _col_spec),
            pl.BlockSpec((DF, DH), lambda i: (0, 0)),
            pl.BlockSpec((_BLK, 1), _col_spec),
            pl.BlockSpec((_BLK, 1), _col_spec),
        ],
        out_specs=pl.BlockSpec((_BLK, DH), _col_spec),
        out_shape=jax.ShapeDtypeStruct((NP, DH), jnp.float32),
    )(xp, W1, d0, d1)


def _tc2(p0, p1, g1, d0, d1, W2, b1r):
    return pl.pallas_call(
        _tc2_body,
        grid=(NP // _BLK,),
        in_specs=[
            pl.BlockSpec((_BLK, DH), _col_spec),
            pl.BlockSpec((_BLK, DH), _col_spec),
            pl.BlockSpec((_BLK, DH), _col_spec),
            pl.BlockSpec((_BLK, 1), _col_spec),
            pl.BlockSpec((_BLK, 1), _col_spec),
            pl.BlockSpec((DH, DO), lambda i: (0, 0)),
            pl.BlockSpec((1, DH), lambda i: (0, 0)),
        ],
        out_specs=pl.BlockSpec((_BLK, DO), _col_spec),
        out_shape=jax.ShapeDtypeStruct((NP, DO), jnp.float32),
    )(p0, p1, g1, d0, d1, W2, b1r)


def _tc3(p0, p1, g2, d0, d1, b2r):
    return pl.pallas_call(
        _tc3_body,
        grid=(NP // _BLK,),
        in_specs=[
            pl.BlockSpec((_BLK, DO), _col_spec),
            pl.BlockSpec((_BLK, DO), _col_spec),
            pl.BlockSpec((_BLK, DO), _col_spec),
            pl.BlockSpec((_BLK, 1), _col_spec),
            pl.BlockSpec((_BLK, 1), _col_spec),
            pl.BlockSpec((1, DO), lambda i: (0, 0)),
        ],
        out_specs=pl.BlockSpec((_BLK, DO), _col_spec),
        out_shape=jax.ShapeDtypeStruct((NP, DO), jnp.float32),
    )(p0, p1, g2, d0, d1, b2r)


def kernel(x, edge_index, W1, b1, W2, b2):
    padn = ER * 128 - E
    pad = (N + (jnp.arange(padn, dtype=jnp.int32) % (NP - N))).astype(jnp.int32)
    src2 = jnp.concatenate([edge_index[0], pad]).reshape(ER, 128)
    dst2 = jnp.concatenate([edge_index[1], pad]).reshape(ER, 128)

    deg0, deg1 = _deg_partials(dst2)
    d0 = deg0.reshape(NP, 1)
    d1 = deg1.reshape(NP, 1)

    xp = jnp.zeros((NP, DF), jnp.float32).at[:N].set(x)
    g1 = _tc1(xp, W1, d0, d1)
    p0, p1 = _scatter_partials(g1, src2, dst2, DH)
    g2 = _tc2(p0, p1, g1, d0, d1, W2, b1.reshape(1, DH))
    q0, q1 = _scatter_partials(g2, src2, dst2, DO)
    out = _tc3(q0, q1, g2, d0, d1, b2.reshape(1, DO))
    return out[:N]


# speedup vs baseline: 52.5735x; 1.4352x over previous
import functools

import jax
import jax.numpy as jnp
from jax import lax
from jax.experimental import pallas as pl
from jax.experimental.pallas import tpu as pltpu
from jax.experimental.pallas import tpu_sc as plsc

N = 10000
NP = 10240
E = 320000
DF = 128
DH = 64
DO = 16
NC, NS = 2, 16
NW = NC * NS
ER = 2560
RPW = ER // NW
CPT = NP // NS

_SC_PARAMS = pltpu.CompilerParams(use_tc_tiling_on_sc=False)


def _mesh():
    return plsc.VectorSubcoreMesh(
        core_axis_name="c", subcore_axis_name="s", num_cores=NC, num_subcores=NS
    )


def _deg_partials(dst2):
    @functools.partial(
        pl.kernel,
        out_type=(
            jax.ShapeDtypeStruct((NP,), jnp.float32),
            jax.ShapeDtypeStruct((NP,), jnp.float32),
        ),
        mesh=_mesh(),
        scratch_types=[
            pltpu.VMEM((RPW, 128), jnp.int32),
            pltpu.VMEM((128,), jnp.float32),
            pltpu.VMEM((CPT,), jnp.float32),
            pltpu.VMEM_SHARED((NP,), jnp.float32),
            pltpu.SemaphoreType.DMA,
        ],
        compiler_params=_SC_PARAMS,
    )
    def kern(dst_hbm, out0_hbm, out1_hbm, didx_v, ones_v, zer_v, deg_sp, sem):
        c = lax.axis_index("c")
        s = lax.axis_index("s")
        w = s * NC + c

        def fill1(i, _):
            ones_v[pl.ds(i * 16, 16)] = jnp.ones((16,), jnp.float32)
            return 0

        lax.fori_loop(0, 128 // 16, fill1, 0)

        def fill0(i, _):
            zer_v[pl.ds(i * 16, 16)] = jnp.zeros((16,), jnp.float32)
            return 0

        lax.fori_loop(0, CPT // 16, fill0, 0)
        pltpu.sync_copy(zer_v, deg_sp.at[pl.ds(s * CPT, CPT)])
        plsc.subcore_barrier()

        pltpu.sync_copy(dst_hbm.at[pl.ds(w * RPW, RPW)], didx_v)

        def body(j, _):
            pltpu.async_copy(ones_v, deg_sp.at[didx_v.at[j]], sem, add=True)
            return 0

        lax.fori_loop(0, RPW, body, 0)

        def drain(j, _):
            pltpu.make_async_copy(ones_v, deg_sp.at[didx_v.at[j]], sem).wait()
            return 0

        lax.fori_loop(0, RPW, drain, 0)
        plsc.subcore_barrier()

        @pl.when(c == 0)
        def _():
            pltpu.sync_copy(deg_sp.at[pl.ds(s * CPT, CPT)], out0_hbm.at[pl.ds(s * CPT, CPT)])

        @pl.when(c == 1)
        def _():
            pltpu.sync_copy(deg_sp.at[pl.ds(s * CPT, CPT)], out1_hbm.at[pl.ds(s * CPT, CPT)])

    return kern(dst2)


def _scatter_partials(g, src2, dst2, d, staged):
    scratch = [
        pltpu.VMEM((RPW, 128), jnp.int32),
        pltpu.VMEM((RPW, 128), jnp.int32),
        [pltpu.VMEM((128, d), jnp.float32)] * 4,
        pltpu.VMEM((64, d), jnp.float32),
        pltpu.VMEM_SHARED((NP, d), jnp.float32),
        (
            pltpu.VMEM_SHARED((NP, d), jnp.float32)
            if staged
            else pltpu.VMEM((8, d), jnp.float32)
        ),
        [pltpu.SemaphoreType.DMA] * 4,
        [pltpu.SemaphoreType.DMA] * 4,
        pltpu.SemaphoreType.DMA,
    ]

    @functools.partial(
        pl.kernel,
        out_type=(
            jax.ShapeDtypeStruct((NP, d), jnp.float32),
            jax.ShapeDtypeStruct((NP, d), jnp.float32),
        ),
        mesh=_mesh(),
        scratch_types=scratch,
        compiler_params=_SC_PARAMS,
    )
    def kern(g_hbm, src_hbm, dst_hbm, out0_hbm, out1_hbm,
             sidx_v, didx_v, rows, zer_v, acc_sp, g_sp,
             semg, sems, semi):
        c = lax.axis_index("c")
        s = lax.axis_index("s")
        w = s * NC + c
        g_src = g_sp if staged else g_hbm

        pltpu.async_copy(src_hbm.at[pl.ds(w * RPW, RPW)], sidx_v, semi)
        pltpu.async_copy(dst_hbm.at[pl.ds(w * RPW, RPW)], didx_v, semi)
        if staged:
            pltpu.sync_copy(g_hbm.at[pl.ds(s * CPT, CPT)], g_sp.at[pl.ds(s * CPT, CPT)])

        def fill0(i, _):
            for jj in range(d // 16):
                zer_v[i, pl.ds(jj * 16, 16)] = jnp.zeros((16,), jnp.float32)
            return 0

        lax.fori_loop(0, 64, fill0, 0)

        def zcp(i, _):
            pltpu.sync_copy(zer_v, acc_sp.at[pl.ds(s * CPT + i * 64, 64)])
            return 0

        lax.fori_loop(0, CPT // 64, zcp, 0)
        pltpu.make_async_copy(src_hbm.at[pl.ds(w * RPW, RPW)], sidx_v, semi).wait()
        pltpu.make_async_copy(dst_hbm.at[pl.ds(w * RPW, RPW)], didx_v, semi).wait()
        plsc.subcore_barrier()

        def gat(j, b):
            pltpu.async_copy(g_src.at[sidx_v.at[j]], rows[b], semg[b])

        def gat_wait(j, b):
            pltpu.make_async_copy(g_src.at[sidx_v.at[j]], rows[b], semg[b]).wait()

        def sca(j, b):
            pltpu.async_copy(rows[b], acc_sp.at[didx_v.at[j]], sems[b], add=True)

        def sca_wait(j, b):
            pltpu.make_async_copy(rows[b], acc_sp.at[didx_v.at[j]], sems[b]).wait()

        gat(0, 0)
        gat(1, 1)
        gat(2, 2)

        def body(i, _):
            for jj in range(4):
                j = 4 * i + jj
                b = jj
                bp = (jj + 3) % 4

                @pl.when(j >= 1)
                def _():
                    sca_wait(j - 1, bp)

                @pl.when(j + 3 < RPW)
                def _():
                    gat(j + 3, bp)

                gat_wait(j, b)
                sca(j, b)
            return 0

        lax.fori_loop(0, RPW // 4, body, 0)
        sca_wait(RPW - 1, (RPW - 1) % 4)
        plsc.subcore_barrier()

        @pl.when(c == 0)
        def _():
            pltpu.sync_copy(acc_sp.at[pl.ds(s * CPT, CPT)], out0_hbm.at[pl.ds(s * CPT, CPT)])

        @pl.when(c == 1)
        def _():
            pltpu.sync_copy(acc_sp.at[pl.ds(s * CPT, CPT)], out1_hbm.at[pl.ds(s * CPT, CPT)])

    return kern(g, src2, dst2)


def _dinv(d0_ref, d1_ref):
    return lax.rsqrt(d0_ref[...] + d1_ref[...] + 1.0)


def _tc1_body(x_ref, w_ref, d0_ref, d1_ref, o_ref):
    dinv = _dinv(d0_ref, d1_ref)
    o_ref[...] = dinv * jnp.dot(
        x_ref[...], w_ref[...], preferred_element_type=jnp.float32
    )


def _tc2_body(p0_ref, p1_ref, g_ref, d0_ref, d1_ref, w_ref, b_ref, o_ref):
    dinv = _dinv(d0_ref, d1_ref)
    a = p0_ref[...] + p1_ref[...] + g_ref[...]
    t = jnp.maximum(dinv * a + b_ref[...], 0.0)
    o_ref[...] = dinv * jnp.dot(t, w_ref[...], preferred_element_type=jnp.float32)


def _tc3_body(p0_ref, p1_ref, g_ref, d0_ref, d1_ref, b_ref, o_ref):
    dinv = _dinv(d0_ref, d1_ref)
    o = dinv * (p0_ref[...] + p1_ref[...] + g_ref[...]) + b_ref[...]
    m = jnp.max(o, axis=1, keepdims=True)
    ex = jnp.exp(o - m)
    lse = jnp.log(jnp.sum(ex, axis=1, keepdims=True))
    o_ref[...] = o - m - lse


_BLK = 1024


def _col_spec(i):
    return (i, 0)


def _tc1(xp, W1, d0, d1):
    return pl.pallas_call(
        _tc1_body,
        grid=(NP // _BLK,),
        in_specs=[
            pl.BlockSpec((_BLK, DF), _col_spec),
            pl.BlockSpec((DF, DH), lambda i: (0, 0)),
            pl.BlockSpec((_BLK, 1), _col_spec),
            pl.BlockSpec((_BLK, 1), _col_spec),
        ],
        out_specs=pl.BlockSpec((_BLK, DH), _col_spec),
        out_shape=jax.ShapeDtypeStruct((NP, DH), jnp.float32),
    )(xp, W1, d0, d1)


def _tc2(p0, p1, g1, d0, d1, W2, b1r):
    return pl.pallas_call(
        _tc2_body,
        grid=(NP // _BLK,),
        in_specs=[
            pl.BlockSpec((_BLK, DH), _col_spec),
            pl.BlockSpec((_BLK, DH), _col_spec),
            pl.BlockSpec((_BLK, DH), _col_spec),
            pl.BlockSpec((_BLK, 1), _col_spec),
            pl.BlockSpec((_BLK, 1), _col_spec),
            pl.BlockSpec((DH, DO), lambda i: (0, 0)),
            pl.BlockSpec((1, DH), lambda i: (0, 0)),
        ],
        out_specs=pl.BlockSpec((_BLK, DO), _col_spec),
        out_shape=jax.ShapeDtypeStruct((NP, DO), jnp.float32),
    )(p0, p1, g1, d0, d1, W2, b1r)


def _tc3(p0, p1, g2, d0, d1, b2r):
    return pl.pallas_call(
        _tc3_body,
        grid=(NP // _BLK,),
        in_specs=[
            pl.BlockSpec((_BLK, DO), _col_spec),
            pl.BlockSpec((_BLK, DO), _col_spec),
            pl.BlockSpec((_BLK, DO), _col_spec),
            pl.BlockSpec((_BLK, 1), _col_spec),
            pl.BlockSpec((_BLK, 1), _col_spec),
            pl.BlockSpec((1, DO), lambda i: (0, 0)),
        ],
        out_specs=pl.BlockSpec((_BLK, DO), _col_spec),
        out_shape=jax.ShapeDtypeStruct((NP, DO), jnp.float32),
    )(p0, p1, g2, d0, d1, b2r)


def kernel(x, edge_index, W1, b1, W2, b2):
    padn = ER * 128 - E
    pad = (N + (jnp.arange(padn, dtype=jnp.int32) % (NP - N))).astype(jnp.int32)
    src2 = jnp.concatenate([edge_index[0], pad]).reshape(ER, 128)
    dst2 = jnp.concatenate([edge_index[1], pad]).reshape(ER, 128)

    deg0, deg1 = _deg_partials(dst2)
    d0 = deg0.reshape(NP, 1)
    d1 = deg1.reshape(NP, 1)

    xp = jnp.zeros((NP, DF), jnp.float32).at[:N].set(x)
    g1 = _tc1(xp, W1, d0, d1)
    p0, p1 = _scatter_partials(g1, src2, dst2, DH, staged=False)
    g2 = _tc2(p0, p1, g1, d0, d1, W2, b1.reshape(1, DH))
    q0, q1 = _scatter_partials(g2, src2, dst2, DO, staged=True)
    out = _tc3(q0, q1, g2, d0, d1, b2.reshape(1, DO))
    return out[:N]
